# Initial kernel scaffold; baseline (speedup 1.0000x reference)
import jax
import jax.numpy as jnp
from jax.experimental import pallas as pl


def kernel(y_a, y_p, targets):
    raise NotImplementedError("write your pallas kernel here")


# speedup vs baseline: 1.4273x; 1.4273x over previous
import functools

import jax
import jax.numpy as jnp
from jax.experimental import pallas as pl

THETA_GLO = 1.15
DELTA = 5
GAMMA = 0.5
EPS = 1e-6

BLOCK_B = 512


def _loss_kernel(ya_ref, yp_ref, ypd_ref, ta_ref, tp_ref, out_ref, *, d, n_rows):
    i = pl.program_id(0)

    a = ya_ref[...]
    p = yp_ref[...]
    p_diag = ypd_ref[...]
    ta = ta_ref[...]
    tp = tp_ref[...]

    a2 = jnp.sum(a * a, axis=1, keepdims=True)
    sa = jnp.sum(a, axis=1, keepdims=True)
    p2 = jnp.sum(p * p, axis=1, keepdims=True).T
    sp = jnp.sum(p, axis=1, keepdims=True).T

    cross = jax.lax.dot_general(
        a, p, (((1,), (1,)), ((), ())), preferred_element_type=jnp.float32)

    d2 = (a2 + p2 - 2.0 * cross
          + (2.0 * EPS) * (sa - sp) + d * EPS * EPS)
    d2 = jnp.where(ta == tp, jnp.inf, d2)
    d_n = jnp.sqrt(jnp.maximum(jnp.min(d2, axis=1, keepdims=True), 0.0))

    diff = a - p_diag + EPS
    d_p = jnp.sqrt(jnp.maximum(jnp.sum(diff * diff, axis=1, keepdims=True), 0.0))

    theta = GAMMA * (d_p + d_n) * 0.5 + (1.0 - GAMMA) * THETA_GLO
    scale = 2.0 * DELTA
    loss = -(jax.nn.log_sigmoid(scale * (theta - d_p))
             + jax.nn.log_sigmoid(scale * (d_n - theta))) / scale

    @pl.when(i == 0)
    def _():
        out_ref[...] = jnp.zeros((1, 1), jnp.float32)

    out_ref[...] += jnp.sum(loss, keepdims=True) / n_rows


def kernel(y_a, y_p, targets):
    b, d = y_a.shape
    targets = targets.astype(jnp.int32)
    t_row = targets.reshape(b, 1)
    t_col = targets.reshape(1, b)
    grid = b // BLOCK_B

    out = pl.pallas_call(
        functools.partial(_loss_kernel, d=d, n_rows=b),
        grid=(grid,),
        in_specs=[
            pl.BlockSpec((BLOCK_B, d), lambda i: (i, 0)),
            pl.BlockSpec((b, d), lambda i: (0, 0)),
            pl.BlockSpec((BLOCK_B, d), lambda i: (i, 0)),
            pl.BlockSpec((BLOCK_B, 1), lambda i: (i, 0)),
            pl.BlockSpec((1, b), lambda i: (0, 0)),
        ],
        out_specs=pl.BlockSpec((1, 1), lambda i: (0, 0)),
        out_shape=jax.ShapeDtypeStruct((1, 1), jnp.float32),
    )(y_a, y_p, y_p, t_row, t_col)

    return out[0, 0]


# speedup vs baseline: 2.4318x; 1.7038x over previous
import functools

import jax
import jax.numpy as jnp
from jax.experimental import pallas as pl

THETA_GLO = 1.15
DELTA = 5
GAMMA = 0.5
EPS = 1e-6

BLOCK_B = 512


def _loss_kernel(ya_ref, yp_ref, ypd_ref, ta_ref, tp_ref, out_ref, *, d, n_rows):
    i = pl.program_id(0)

    a = ya_ref[...]
    p = yp_ref[...]
    p_diag = ypd_ref[...]
    ta = ta_ref[...]
    tp = tp_ref[...]

    r_a = jnp.sum(a * a + (2.0 * EPS) * a, axis=1, keepdims=True)
    c_p = (jnp.sum(p * p - (2.0 * EPS) * p, axis=1, keepdims=True).T
           + d * EPS * EPS)

    cross = jax.lax.dot_general(
        a, p, (((1,), (1,)), ((), ())), preferred_element_type=jnp.float32)

    e = jnp.where(ta == tp, jnp.inf, c_p - 2.0 * cross)
    m = jnp.min(e, axis=1, keepdims=True) + r_a
    d_n = jnp.sqrt(jnp.maximum(m, 0.0))

    diff = a - p_diag + EPS
    d_p = jnp.sqrt(jnp.maximum(jnp.sum(diff * diff, axis=1, keepdims=True), 0.0))

    theta = GAMMA * (d_p + d_n) * 0.5 + (1.0 - GAMMA) * THETA_GLO
    scale = 2.0 * DELTA
    loss = -(jax.nn.log_sigmoid(scale * (theta - d_p))
             + jax.nn.log_sigmoid(scale * (d_n - theta))) / scale

    @pl.when(i == 0)
    def _():
        out_ref[...] = jnp.zeros((1, 1), jnp.float32)

    out_ref[...] += jnp.sum(loss, keepdims=True) / n_rows


def kernel(y_a, y_p, targets):
    b, d = y_a.shape
    targets = targets.astype(jnp.int32)
    t_row = targets.reshape(b, 1)
    t_col = targets.reshape(1, b)
    grid = b // BLOCK_B

    out = pl.pallas_call(
        functools.partial(_loss_kernel, d=d, n_rows=b),
        grid=(grid,),
        in_specs=[
            pl.BlockSpec((BLOCK_B, d), lambda i: (i, 0)),
            pl.BlockSpec((b, d), lambda i: (0, 0)),
            pl.BlockSpec((BLOCK_B, d), lambda i: (i, 0)),
            pl.BlockSpec((BLOCK_B, 1), lambda i: (i, 0)),
            pl.BlockSpec((1, b), lambda i: (0, 0)),
        ],
        out_specs=pl.BlockSpec((1, 1), lambda i: (0, 0)),
        out_shape=jax.ShapeDtypeStruct((1, 1), jnp.float32),
    )(y_a, y_p, y_p, t_row, t_col)

    return out[0, 0]


# speedup vs baseline: 3.2571x; 1.3394x over previous
import functools

import jax
import jax.numpy as jnp
from jax.experimental import pallas as pl
from jax.experimental.pallas import tpu as pltpu

THETA_GLO = 1.15
DELTA = 5
GAMMA = 0.5
EPS = 1e-6

BLOCK_B = 512


def _loss_kernel(ya_ref, yp_ref, ypd_ref, ta_ref, tp_ref, out_ref, cp_ref,
                 *, d, n_rows):
    i = pl.program_id(0)

    a = ya_ref[...]
    p = yp_ref[...]
    p_diag = ypd_ref[...]
    ta = ta_ref[...]
    tp = tp_ref[...]

    @pl.when(i == 0)
    def _():
        cp_ref[...] = (jnp.sum(p * p - (2.0 * EPS) * p, axis=1,
                               keepdims=True).T + d * EPS * EPS)

    c_p = cp_ref[...]
    r_a = jnp.sum(a * a + (2.0 * EPS) * a, axis=1, keepdims=True)

    cross_m = jax.lax.dot_general(
        -2.0 * a, p, (((1,), (1,)), ((), ())),
        preferred_element_type=jnp.float32)

    e = jnp.where(ta == tp, jnp.inf, c_p + cross_m)
    m = jnp.min(e, axis=1, keepdims=True) + r_a
    d_n = jnp.sqrt(jnp.maximum(m, 0.0))

    diff = a - p_diag + EPS
    d_p = jnp.sqrt(jnp.maximum(jnp.sum(diff * diff, axis=1, keepdims=True), 0.0))

    theta = GAMMA * (d_p + d_n) * 0.5 + (1.0 - GAMMA) * THETA_GLO
    scale = 2.0 * DELTA
    loss = -(jax.nn.log_sigmoid(scale * (theta - d_p))
             + jax.nn.log_sigmoid(scale * (d_n - theta))) / scale

    @pl.when(i == 0)
    def _():
        out_ref[...] = jnp.zeros((1, 1), jnp.float32)

    out_ref[...] += jnp.sum(loss, keepdims=True) / n_rows


def kernel(y_a, y_p, targets):
    b, d = y_a.shape
    targets = targets.astype(jnp.int32)
    t_row = targets.reshape(b, 1)
    t_col = targets.reshape(1, b)
    grid = b // BLOCK_B

    out = pl.pallas_call(
        functools.partial(_loss_kernel, d=d, n_rows=b),
        grid=(grid,),
        in_specs=[
            pl.BlockSpec((BLOCK_B, d), lambda i: (i, 0)),
            pl.BlockSpec((b, d), lambda i: (0, 0)),
            pl.BlockSpec((BLOCK_B, d), lambda i: (i, 0)),
            pl.BlockSpec((BLOCK_B, 1), lambda i: (i, 0)),
            pl.BlockSpec((1, b), lambda i: (0, 0)),
        ],
        out_specs=pl.BlockSpec((1, 1), lambda i: (0, 0)),
        out_shape=jax.ShapeDtypeStruct((1, 1), jnp.float32),
        scratch_shapes=[pltpu.VMEM((1, b), jnp.float32)],
    )(y_a, y_p, y_p, t_row, t_col)

    return out[0, 0]
